# baseline (speedup 1.0000x reference)
import functools

import jax
import jax.numpy as jnp
from jax import lax
from jax.experimental import pallas as pl
from jax.experimental.pallas import tpu as pltpu
from jax.experimental.pallas import tpu_sc as plsc

W = 1_000_000
C = 64
N = 16_384
L = 16

_info = plsc.get_sparse_core_info()
NC = _info.num_cores
NS = _info.num_subcores
NW = NC * NS
BPW = N // NW
ICH = 128
NCHUNK = BPW // ICH


def _body(data_hbm, param_hbm, out_hbm,
          param_v, w_v, idx0_v, idx1_v, rows0_v, rows1_v, sem):
    wid = lax.axis_index("s") * NC + lax.axis_index("c")
    base = wid * BPW

    pltpu.sync_copy(param_hbm.at[pl.ds(base, BPW)], param_v)

    for i in range(BPW // L):
        p = param_v[pl.ds(i * L, L)]
        x = jnp.minimum(jnp.maximum(p, 0.0), 1.0) * float(W - 1)
        x0i = x.astype(jnp.int32)
        w = x - x0i.astype(jnp.float32)
        x1i = jnp.minimum(x0i + 1, W - 1)
        idx0_v[i // (ICH // L), pl.ds((i % (ICH // L)) * L, L)] = x0i
        idx1_v[i // (ICH // L), pl.ds((i % (ICH // L)) * L, L)] = x1i
        w_v[pl.ds(i * L, L)] = w

    copies = []
    for j in range(NCHUNK):
        copies.append(pltpu.make_async_copy(
            data_hbm.at[idx0_v.at[j]], rows0_v.at[pl.ds(j * ICH, ICH)], sem))
        copies.append(pltpu.make_async_copy(
            data_hbm.at[idx1_v.at[j]], rows1_v.at[pl.ds(j * ICH, ICH)], sem))
    for cp in copies:
        cp.start()
    for cp in copies:
        cp.wait()

    def grp_body(g, carry):
        w16 = w_v[pl.ds(g * L, L)]
        om16 = 1.0 - w16
        for j in range(L):
            w = w16[j]
            om = om16[j]
            n = g * L + j
            for c in range(C // L):
                sl = pl.ds(c * L, L)
                r0 = rows0_v[n, sl]
                r1 = rows1_v[n, sl]
                rows0_v[n, sl] = r0 * om + r1 * w
        return carry
    lax.fori_loop(0, BPW // L, grp_body, 0)

    pltpu.sync_copy(rows0_v, out_hbm.at[pl.ds(base, BPW)])


@functools.partial(
    pl.kernel,
    mesh=plsc.VectorSubcoreMesh(core_axis_name="c", subcore_axis_name="s"),
    out_type=jax.ShapeDtypeStruct((N, C), jnp.float32),
    compiler_params=pltpu.CompilerParams(use_tc_tiling_on_sc=False),
    scratch_types=[
        pltpu.VMEM((BPW,), jnp.float32),
        pltpu.VMEM((BPW,), jnp.float32),
        pltpu.VMEM((NCHUNK, ICH), jnp.int32),
        pltpu.VMEM((NCHUNK, ICH), jnp.int32),
        pltpu.VMEM((BPW, C), jnp.float32),
        pltpu.VMEM((BPW, C), jnp.float32),
        pltpu.SemaphoreType.DMA,
    ],
)
def _sampler(data_hbm, param_hbm, out_hbm, *scratch):
    _body(data_hbm, param_hbm, out_hbm, *scratch)


def kernel(data, param):
    return _sampler(data, param)


# speedup vs baseline: 1.4703x; 1.4703x over previous
import functools

import jax
import jax.numpy as jnp
from jax import lax
from jax.experimental import pallas as pl
from jax.experimental.pallas import tpu as pltpu
from jax.experimental.pallas import tpu_sc as plsc

W = 1_000_000
C = 64
N = 16_384
L = 16
R = 16

_info = plsc.get_sparse_core_info()
NC = _info.num_cores
NS = _info.num_subcores
NW = NC * NS
BPW = N // NW
CHUNK = 32
NCHUNKS = BPW // CHUNK


def _body(data_hbm, param_hbm, out_hbm,
          param_v, w_v, st_v, o0_v, o1_v, win_v, out_c, sem):
    wid = lax.axis_index("s") * NC + lax.axis_index("c")
    base = wid * BPW

    pltpu.sync_copy(param_hbm.at[pl.ds(base, BPW)], param_v)

    for i in range(BPW // L):
        p = param_v[pl.ds(i * L, L)]
        x = jnp.minimum(jnp.maximum(p, 0.0), 1.0) * float(W - 1)
        x0i = x.astype(jnp.int32)
        w = x - x0i.astype(jnp.float32)
        x1i = jnp.minimum(x0i + 1, W - 1)
        st = jnp.minimum((x0i >> 3) << 3, W - R)
        sl = pl.ds(i * L, L)
        w_v[sl] = w
        st_v[sl] = st
        o0_v[sl] = x0i - st
        o1_v[sl] = x1i - st

    def chunk_body(k, carry):
        cbase = k * CHUNK
        for g in range(CHUNK // L):
            st16 = st_v[pl.ds(cbase + g * L, L)]
            for j in range(L):
                pltpu.make_async_copy(
                    data_hbm.at[pl.ds(pl.multiple_of(st16[j], 8), R)],
                    win_v.at[g * L + j], sem).start()

        def drain(r, c):
            pltpu.make_async_copy(
                data_hbm.at[pl.ds(0, R)], win_v.at[r], sem).wait()
            return c
        lax.fori_loop(0, CHUNK, drain, 0)

        for g in range(CHUNK // L):
            gsl = pl.ds(cbase + g * L, L)
            w16 = w_v[gsl]
            om16 = 1.0 - w16
            o016 = o0_v[gsl]
            o116 = o1_v[gsl]
            for j in range(L):
                r = g * L + j
                w = w16[j]
                om = om16[j]
                o0 = o016[j]
                o1 = o116[j]
                for c in range(C // L):
                    csl = pl.ds(c * L, L)
                    out_c[r, csl] = win_v[r, o0, csl] * om + win_v[r, o1, csl] * w
        pltpu.sync_copy(
            out_c, out_hbm.at[pl.ds(pl.multiple_of(base + cbase, 8), CHUNK)])
        return carry
    lax.fori_loop(0, NCHUNKS, chunk_body, 0)


@functools.partial(
    pl.kernel,
    mesh=plsc.VectorSubcoreMesh(core_axis_name="c", subcore_axis_name="s"),
    out_type=jax.ShapeDtypeStruct((N, C), jnp.float32),
    scratch_types=[
        pltpu.VMEM((BPW,), jnp.float32),
        pltpu.VMEM((BPW,), jnp.float32),
        pltpu.VMEM((BPW,), jnp.int32),
        pltpu.VMEM((BPW,), jnp.int32),
        pltpu.VMEM((BPW,), jnp.int32),
        pltpu.VMEM((CHUNK, R, C), jnp.float32),
        pltpu.VMEM((CHUNK, C), jnp.float32),
        pltpu.SemaphoreType.DMA,
    ],
)
def _sampler(data_hbm, param_hbm, out_hbm, *scratch):
    _body(data_hbm, param_hbm, out_hbm, *scratch)


def kernel(data, param):
    return _sampler(data, param)
